# Initial kernel scaffold; baseline (speedup 1.0000x reference)
import jax
import jax.numpy as jnp
from jax.experimental import pallas as pl


def kernel(x, table):
    raise NotImplementedError("write your pallas kernel here")


# speedup vs baseline: 1.1516x; 1.1516x over previous
import functools

import jax
import jax.numpy as jnp
from jax import lax
from jax.experimental import pallas as pl
from jax.experimental.pallas import tpu as pltpu
from jax.experimental.pallas import tpu_sc as plsc

VOCAB = 1000000
EMBED_DIM = 32
BATCH = 16384
HIST = 50
TOTAL = BATCH * HIST

_info = plsc.get_sparse_core_info()
NUM_CORES = _info.num_cores
NUM_SUBCORES = _info.num_subcores
NUM_WORKERS = NUM_CORES * NUM_SUBCORES

B_PER_W = TOTAL // NUM_WORKERS
CHUNK = 1600
N_CHUNKS = B_PER_W // CHUNK

_mesh = plsc.VectorSubcoreMesh(core_axis_name="c", subcore_axis_name="s")


@functools.partial(
    pl.kernel,
    mesh=_mesh,
    compiler_params=pltpu.CompilerParams(use_tc_tiling_on_sc=False),
    out_type=jax.ShapeDtypeStruct((TOTAL, EMBED_DIM), jnp.float32),
    scratch_types=[
        pltpu.VMEM((CHUNK,), jnp.int32),
        pltpu.VMEM((CHUNK, EMBED_DIM), jnp.float32),
        pltpu.SemaphoreType.DMA,
    ],
)
def _gather_kernel(idx_hbm, table_hbm, out_hbm, idx_v, rows_v, sem):
    wid = lax.axis_index("s") * NUM_CORES + lax.axis_index("c")
    wbase = wid * B_PER_W

    def body(c, carry):
        base = wbase + c * CHUNK
        pltpu.sync_copy(idx_hbm.at[pl.ds(base, CHUNK)], idx_v)
        pltpu.async_copy(table_hbm.at[idx_v], rows_v, sem).wait()
        pltpu.sync_copy(rows_v, out_hbm.at[pl.ds(base, CHUNK)])
        return carry

    lax.fori_loop(0, N_CHUNKS, body, 0)


def kernel(x, table):
    idx = x.reshape(TOTAL).astype(jnp.int32)
    out = _gather_kernel(idx, table)
    return out.reshape(BATCH, HIST, EMBED_DIM)


# speedup vs baseline: 1.1624x; 1.0094x over previous
import functools

import jax
import jax.numpy as jnp
from jax import lax
from jax.experimental import pallas as pl
from jax.experimental.pallas import tpu as pltpu
from jax.experimental.pallas import tpu_sc as plsc

VOCAB = 1000000
EMBED_DIM = 32
BATCH = 16384
HIST = 50
TOTAL = BATCH * HIST

_info = plsc.get_sparse_core_info()
NUM_CORES = _info.num_cores
NUM_SUBCORES = _info.num_subcores
NUM_WORKERS = NUM_CORES * NUM_SUBCORES

B_PER_W = TOTAL // NUM_WORKERS
CHUNK = 1600
N_CHUNKS = B_PER_W // CHUNK
NBUF = 2

_mesh = plsc.VectorSubcoreMesh(core_axis_name="c", subcore_axis_name="s")


@functools.partial(
    pl.kernel,
    mesh=_mesh,
    compiler_params=pltpu.CompilerParams(use_tc_tiling_on_sc=False),
    out_type=jax.ShapeDtypeStruct((TOTAL, EMBED_DIM), jnp.float32),
    scratch_types=[
        pltpu.VMEM((B_PER_W,), jnp.int32),
        pltpu.VMEM((CHUNK, EMBED_DIM), jnp.float32),
        pltpu.VMEM((CHUNK, EMBED_DIM), jnp.float32),
        pltpu.SemaphoreType.DMA,
        pltpu.SemaphoreType.DMA,
        pltpu.SemaphoreType.DMA,
        pltpu.SemaphoreType.DMA,
    ],
)
def _gather_kernel(idx_hbm, table_hbm, out_hbm, idx_all, rows0, rows1,
                   g0, g1, s0, s1):
    wid = lax.axis_index("s") * NUM_CORES + lax.axis_index("c")
    wbase = wid * B_PER_W

    row_bufs = (rows0, rows1)
    g_sems = (g0, g1)
    s_sems = (s0, s1)

    pltpu.sync_copy(idx_hbm.at[pl.ds(wbase, B_PER_W)], idx_all)

    def start_gather(c):
        b = c % NBUF
        return pltpu.async_copy(
            table_hbm.at[idx_all.at[pl.ds(c * CHUNK, CHUNK)]],
            row_bufs[b], g_sems[b])

    gathers = {}
    stores = {}
    for c in range(min(NBUF, N_CHUNKS)):
        gathers[c] = start_gather(c)
    for c in range(N_CHUNKS):
        b = c % NBUF
        gathers[c].wait()
        stores[c] = pltpu.async_copy(
            row_bufs[b], out_hbm.at[pl.ds(wbase + c * CHUNK, CHUNK)],
            s_sems[b])
        nc = c + NBUF
        if nc < N_CHUNKS:
            stores[c].wait()
            gathers[nc] = start_gather(nc)
    for c in range(max(0, N_CHUNKS - NBUF), N_CHUNKS):
        stores[c].wait()


def kernel(x, table):
    idx = x.reshape(TOTAL).astype(jnp.int32)
    out = _gather_kernel(idx, table)
    return out.reshape(BATCH, HIST, EMBED_DIM)


# speedup vs baseline: 1.8841x; 1.6209x over previous
import functools

import jax
import jax.numpy as jnp
from jax import lax
from jax.experimental import pallas as pl
from jax.experimental.pallas import tpu as pltpu
from jax.experimental.pallas import tpu_sc as plsc

VOCAB = 1000000
EMBED_DIM = 32
BATCH = 16384
HIST = 50

_info = plsc.get_sparse_core_info()
NUM_CORES = _info.num_cores
NUM_SUBCORES = _info.num_subcores
NUM_WORKERS = NUM_CORES * NUM_SUBCORES

ROWS_PER_W = BATCH // NUM_WORKERS
RCHUNK = 32
N_CHUNKS = ROWS_PER_W // RCHUNK
NBUF = 2

_mesh = plsc.VectorSubcoreMesh(core_axis_name="c", subcore_axis_name="s")


@functools.partial(
    pl.kernel,
    mesh=_mesh,
    compiler_params=pltpu.CompilerParams(use_tc_tiling_on_sc=False),
    out_type=jax.ShapeDtypeStruct((BATCH, HIST, EMBED_DIM), jnp.float32),
    scratch_types=[
        pltpu.VMEM((ROWS_PER_W, HIST), jnp.int32),
        pltpu.VMEM((RCHUNK, HIST, EMBED_DIM), jnp.float32),
        pltpu.VMEM((RCHUNK, HIST, EMBED_DIM), jnp.float32),
        pltpu.SemaphoreType.DMA,
        pltpu.SemaphoreType.DMA,
        pltpu.SemaphoreType.DMA,
        pltpu.SemaphoreType.DMA,
    ],
)
def _gather_kernel(idx_hbm, table_hbm, out_hbm, idx_all, rows0, rows1,
                   g0, g1, s0, s1):
    wid = lax.axis_index("s") * NUM_CORES + lax.axis_index("c")
    wbase = wid * ROWS_PER_W

    row_bufs = (rows0, rows1)
    g_sems = (g0, g1)
    s_sems = (s0, s1)

    pltpu.sync_copy(idx_hbm.at[pl.ds(wbase, ROWS_PER_W), :], idx_all)

    def start_gather(c):
        b = c % NBUF

        def issue_row(r, carry):
            pltpu.async_copy(
                table_hbm.at[idx_all.at[c * RCHUNK + r]],
                row_bufs[b].at[r], g_sems[b])
            return carry

        lax.fori_loop(0, RCHUNK, issue_row, 0)

    def wait_gather(c):
        b = c % NBUF

        def wait_row(r, carry):
            pltpu.make_async_copy(
                table_hbm.at[idx_all.at[c * RCHUNK + r]],
                row_bufs[b].at[r], g_sems[b]).wait()
            return carry

        lax.fori_loop(0, RCHUNK, wait_row, 0)

    stores = {}
    for c in range(min(NBUF, N_CHUNKS)):
        start_gather(c)
    for c in range(N_CHUNKS):
        b = c % NBUF
        wait_gather(c)
        stores[c] = pltpu.async_copy(
            row_bufs[b],
            out_hbm.at[pl.ds(wbase + c * RCHUNK, RCHUNK), :, :],
            s_sems[b])
        nc = c + NBUF
        if nc < N_CHUNKS:
            stores[c].wait()
            start_gather(nc)
    for c in range(max(0, N_CHUNKS - NBUF), N_CHUNKS):
        stores[c].wait()


def kernel(x, table):
    return _gather_kernel(x.astype(jnp.int32), table)


# speedup vs baseline: 2.6440x; 1.4033x over previous
import functools

import jax
import jax.numpy as jnp
from jax import lax
from jax.experimental import pallas as pl
from jax.experimental.pallas import tpu as pltpu
from jax.experimental.pallas import tpu_sc as plsc

VOCAB = 1000000
EMBED_DIM = 32
PAD_DIM = 128
BATCH = 16384
HIST = 50
HIST_PAD = 56

_info = plsc.get_sparse_core_info()
NUM_CORES = _info.num_cores
NUM_SUBCORES = _info.num_subcores
NUM_WORKERS = NUM_CORES * NUM_SUBCORES

ROWS_PER_W = BATCH // NUM_WORKERS
RCHUNK = 32
N_CHUNKS = ROWS_PER_W // RCHUNK
NBUF = 2

_mesh = plsc.VectorSubcoreMesh(core_axis_name="c", subcore_axis_name="s")


@functools.partial(
    pl.kernel,
    mesh=_mesh,
    compiler_params=pltpu.CompilerParams(use_tc_tiling_on_sc=False),
    out_type=jax.ShapeDtypeStruct((BATCH, HIST_PAD, PAD_DIM), jnp.float32),
    scratch_types=[
        pltpu.VMEM((ROWS_PER_W, HIST), jnp.int32),
        pltpu.VMEM((RCHUNK, HIST, EMBED_DIM), jnp.float32),
        pltpu.VMEM((RCHUNK, HIST, EMBED_DIM), jnp.float32),
        pltpu.SemaphoreType.DMA,
        pltpu.SemaphoreType.DMA,
        pltpu.SemaphoreType.DMA,
        pltpu.SemaphoreType.DMA,
    ],
)
def _gather_kernel(idx_hbm, table_hbm, out_hbm, idx_all, rows0, rows1,
                   g0, g1, s0, s1):
    wid = lax.axis_index("s") * NUM_CORES + lax.axis_index("c")
    wbase = wid * ROWS_PER_W

    row_bufs = (rows0, rows1)
    g_sems = (g0, g1)
    s_sems = (s0, s1)

    pltpu.sync_copy(idx_hbm.at[pl.ds(wbase, ROWS_PER_W), :], idx_all)

    def start_gather(c):
        b = c % NBUF

        def issue_row(r, carry):
            pltpu.async_copy(
                table_hbm.at[idx_all.at[c * RCHUNK + r]],
                row_bufs[b].at[r], g_sems[b])
            return carry

        lax.fori_loop(0, RCHUNK, issue_row, 0)

    def wait_gather(c):
        b = c % NBUF

        def wait_row(r, carry):
            pltpu.make_async_copy(
                table_hbm.at[idx_all.at[c * RCHUNK + r]],
                row_bufs[b].at[r], g_sems[b]).wait()
            return carry

        lax.fori_loop(0, RCHUNK, wait_row, 0)

    stores = {}
    for c in range(min(NBUF, N_CHUNKS)):
        start_gather(c)
    for c in range(N_CHUNKS):
        b = c % NBUF
        wait_gather(c)
        stores[c] = pltpu.async_copy(
            row_bufs[b],
            out_hbm.at[pl.ds(wbase + c * RCHUNK, RCHUNK),
                       pl.ds(0, HIST), pl.ds(0, EMBED_DIM)],
            s_sems[b])
        nc = c + NBUF
        if nc < N_CHUNKS:
            stores[c].wait()
            start_gather(nc)
    for c in range(max(0, N_CHUNKS - NBUF), N_CHUNKS):
        stores[c].wait()


def kernel(x, table):
    out_p = _gather_kernel(x.astype(jnp.int32), table)
    return out_p[:, :HIST, :EMBED_DIM]
